# baseline (speedup 1.0000x reference)
import jax
import jax.numpy as jnp
from jax.experimental import pallas as pl

INPUT_DIM = 768
OUTPUT_DIM = 768
LATENT_DIM = 4096
TOPK = 128
GAMMA = 1.0
BETA = 1.0

BLOCK_M = 256
NPACK = 32
NGROUP = LATENT_DIM // NPACK


def _bit_transpose32(w):
    w = list(w)
    masks = {16: 0x0000FFFF, 8: 0x00FF00FF, 4: 0x0F0F0F0F,
             2: 0x33333333, 1: 0x55555555}
    for j in (16, 8, 4, 2, 1):
        m = jnp.int32(masks[j])
        k = 0
        while k < 32:
            a, b = w[k], w[k + j]
            t = (jax.lax.shift_right_logical(b, jnp.int32(j)) ^ a) & m
            w[k] = a ^ t
            w[k + j] = b ^ jax.lax.shift_left(t, jnp.int32(j))
            k = (k + j + 1) & ~j
    return [w[31 - b] for b in range(32)]


def _body(hT_ref, weT_ref, beT_ref, wd_ref, bd_ref, out_ref, zs_ref):
    hT = hT_ref[...]
    preT = jnp.dot(weT_ref[...], hT, preferred_element_type=jnp.float32)
    preT = preT + beT_ref[...]
    zT = jnp.maximum(preT, 0.0) + jnp.where(preT > GAMMA, BETA, 0.0)

    zbT = jax.lax.bitcast_convert_type(zT, jnp.int32)
    w = [zbT[i * NGROUP:(i + 1) * NGROUP, :] for i in range(NPACK)]
    planes = _bit_transpose32(w)

    active = jnp.full((NGROUP, BLOCK_M), -1, jnp.int32)
    cnt_above = jnp.zeros((1, BLOCK_M), jnp.int32)
    t = jnp.zeros((1, BLOCK_M), jnp.int32)
    for b in range(30, -1, -1):
        ones = active & planes[b]
        n1 = jnp.sum(jax.lax.population_count(ones), axis=0, keepdims=True)
        take = (cnt_above + n1) >= TOPK
        t = jnp.where(take, t | (1 << b), t)
        active = jnp.where(take, ones, active ^ ones)
        cnt_above = jnp.where(take, cnt_above, cnt_above + n1)
    thrT = jax.lax.bitcast_convert_type(t, jnp.float32)

    zsT = jnp.where(zT >= thrT, zT, 0.0)
    zs = zsT.T
    zs_ref[...] = zs
    out_ref[...] = (
        jnp.dot(zs, wd_ref[...], preferred_element_type=jnp.float32)
        + bd_ref[...]
    )


@jax.jit
def kernel(h_2, W_enc, b_enc, W_dec, b_dec):
    n = h_2.shape[0]
    grid = (n // BLOCK_M,)
    hT_full = h_2.T
    weT = W_enc.T
    beT = b_enc.reshape(LATENT_DIM, 1)
    bd = b_dec.reshape(1, OUTPUT_DIM)
    h_1_recon, z_sparse = pl.pallas_call(
        _body,
        grid=grid,
        in_specs=[
            pl.BlockSpec((INPUT_DIM, BLOCK_M), lambda i: (0, i)),
            pl.BlockSpec((LATENT_DIM, INPUT_DIM), lambda i: (0, 0)),
            pl.BlockSpec((LATENT_DIM, 1), lambda i: (0, 0)),
            pl.BlockSpec((LATENT_DIM, OUTPUT_DIM), lambda i: (0, 0)),
            pl.BlockSpec((1, OUTPUT_DIM), lambda i: (0, 0)),
        ],
        out_specs=[
            pl.BlockSpec((BLOCK_M, OUTPUT_DIM), lambda i: (i, 0)),
            pl.BlockSpec((BLOCK_M, LATENT_DIM), lambda i: (i, 0)),
        ],
        out_shape=[
            jax.ShapeDtypeStruct((n, OUTPUT_DIM), jnp.float32),
            jax.ShapeDtypeStruct((n, LATENT_DIM), jnp.float32),
        ],
    )(hT_full, weT, beT, W_dec, bd)
    return (h_1_recon, z_sparse)


# speedup vs baseline: 1.3200x; 1.3200x over previous
import jax
import jax.numpy as jnp
from jax.experimental import pallas as pl

INPUT_DIM = 768
OUTPUT_DIM = 768
LATENT_DIM = 4096
TOPK = 128
GAMMA = 1.0
BETA = 1.0

BLOCK_M = 256
RGROUP = 8
NPACK = 32
NGROUP = LATENT_DIM // NPACK


def _bit_transpose32(w):
    w = list(w)
    masks = {16: 0x0000FFFF, 8: 0x00FF00FF, 4: 0x0F0F0F0F,
             2: 0x33333333, 1: 0x55555555}
    for j in (16, 8, 4, 2, 1):
        m = jnp.int32(masks[j])
        k = 0
        while k < 32:
            a, b = w[k], w[k + j]
            t = (jax.lax.shift_right_logical(b, jnp.int32(j)) ^ a) & m
            w[k] = a ^ t
            w[k + j] = b ^ jax.lax.shift_left(t, jnp.int32(j))
            k = (k + j + 1) & ~j
    return [w[31 - b] for b in range(32)]


def _group_threshold(s):
    r = s.shape[0]
    w = [s[:, i * NGROUP:(i + 1) * NGROUP] for i in range(NPACK)]
    planes = _bit_transpose32(w)
    active = jnp.full((r, NGROUP), -1, jnp.int32)
    cnt_above = jnp.zeros((r, 1), jnp.float32)
    t = jnp.zeros((r, 1), jnp.int32)
    for b in range(30, -1, -1):
        ones = active & planes[b]
        pc = jax.lax.population_count(ones).astype(jnp.float32)
        n1 = jnp.sum(pc, axis=1, keepdims=True)
        take = (cnt_above + n1) >= TOPK
        t = jnp.where(take, t | (1 << b), t)
        active = jnp.where(take, ones, active ^ ones)
        cnt_above = jnp.where(take, cnt_above, cnt_above + n1)
    return t


def _body(h_ref, we_ref, be_ref, wd_ref, bd_ref, out_ref, zs_ref):
    h = h_ref[...]
    pre = jnp.dot(h, we_ref[...], preferred_element_type=jnp.float32)
    pre = pre + be_ref[...]
    z = jnp.maximum(pre, 0.0) + jnp.where(pre > GAMMA, BETA, 0.0)

    zb = jax.lax.bitcast_convert_type(z, jnp.int32)
    t = _group_threshold(zb)
    thr = jax.lax.bitcast_convert_type(t, jnp.float32)

    zs = jnp.where(z >= thr, z, 0.0)
    zs_ref[...] = zs
    out_ref[...] = (
        jnp.dot(zs, wd_ref[...], preferred_element_type=jnp.float32)
        + bd_ref[...]
    )


@jax.jit
def kernel(h_2, W_enc, b_enc, W_dec, b_dec):
    n = h_2.shape[0]
    grid = (n // BLOCK_M,)
    be = b_enc.reshape(1, LATENT_DIM)
    bd = b_dec.reshape(1, OUTPUT_DIM)
    h_1_recon, z_sparse = pl.pallas_call(
        _body,
        grid=grid,
        in_specs=[
            pl.BlockSpec((BLOCK_M, INPUT_DIM), lambda i: (i, 0)),
            pl.BlockSpec((INPUT_DIM, LATENT_DIM), lambda i: (0, 0)),
            pl.BlockSpec((1, LATENT_DIM), lambda i: (0, 0)),
            pl.BlockSpec((LATENT_DIM, OUTPUT_DIM), lambda i: (0, 0)),
            pl.BlockSpec((1, OUTPUT_DIM), lambda i: (0, 0)),
        ],
        out_specs=[
            pl.BlockSpec((BLOCK_M, OUTPUT_DIM), lambda i: (i, 0)),
            pl.BlockSpec((BLOCK_M, LATENT_DIM), lambda i: (i, 0)),
        ],
        out_shape=[
            jax.ShapeDtypeStruct((n, OUTPUT_DIM), jnp.float32),
            jax.ShapeDtypeStruct((n, LATENT_DIM), jnp.float32),
        ],
    )(h_2, W_enc, be, W_dec, bd)
    return (h_1_recon, z_sparse)
